# Initial kernel scaffold; baseline (speedup 1.0000x reference)
import functools

import jax
import jax.numpy as jnp
from jax import lax
from jax.experimental import pallas as pl
from jax.experimental.pallas import tpu as pltpu
from jax.experimental.pallas import tpu_sc as plsc

N = 10000
E = 320000
D_IN = 128
HID = 128
OUT = 64

NC = 2
NS = 16
NW = NC * NS
EW = E // NW
K = 80
NCHUNK = EW // K
RPT = 632
NPAD = RPT * NS

_BR = 1000
_GRID = N // _BR


def _dot(a, b):
    return jnp.dot(a, b, precision=lax.Precision.HIGHEST,
                   preferred_element_type=jnp.float32)


def _stage_a_body(x_ref, w_ref, at_s_ref, at_d_ref, h_ref, av_ref, bv_ref):
    h = _dot(x_ref[...], w_ref[...])
    h_ref[...] = h
    av_ref[...] = _dot(h, at_s_ref[...])
    bv_ref[...] = _dot(h, at_d_ref[...])


def _stage_a(x, W1, at_s, at_d):
    return pl.pallas_call(
        _stage_a_body,
        grid=(_GRID,),
        in_specs=[
            pl.BlockSpec((_BR, D_IN), lambda i: (i, 0)),
            pl.BlockSpec((D_IN, HID), lambda i: (0, 0)),
            pl.BlockSpec((HID, 1), lambda i: (0, 0)),
            pl.BlockSpec((HID, 1), lambda i: (0, 0)),
        ],
        out_specs=[
            pl.BlockSpec((_BR, HID), lambda i: (i, 0)),
            pl.BlockSpec((_BR, 1), lambda i: (i, 0)),
            pl.BlockSpec((_BR, 1), lambda i: (i, 0)),
        ],
        out_shape=[
            jax.ShapeDtypeStruct((N, HID), jnp.float32),
            jax.ShapeDtypeStruct((N, 1), jnp.float32),
            jax.ShapeDtypeStruct((N, 1), jnp.float32),
        ],
    )(x, W1, at_s, at_d)


def _stage_b_body(p0_ref, p1_ref, s0_ref, s1_ref, av_ref, bv_ref, h1_ref,
                  b1_ref, w2_ref, a2s_ref, a2d_ref, h2_ref, av2_ref, bv2_ref):
    e = av_ref[...] + bv_ref[...]
    w = jnp.exp(jnp.maximum(e, 0.2 * e))
    num = p0_ref[...] + p1_ref[...] + w * h1_ref[...]
    den = s0_ref[...] + s1_ref[...] + w
    agg = num / den + b1_ref[...]
    x2 = jnp.where(agg > 0, agg, jnp.expm1(agg))
    h2 = _dot(x2, w2_ref[...])
    h2_ref[...] = h2
    av2_ref[...] = _dot(h2, a2s_ref[...])
    bv2_ref[...] = _dot(h2, a2d_ref[...])


def _stage_b(p0, p1, s0, s1, av, bv, h1, b1r, W2, a2s, a2d):
    return pl.pallas_call(
        _stage_b_body,
        grid=(_GRID,),
        in_specs=[
            pl.BlockSpec((_BR, HID), lambda i: (i, 0)),
            pl.BlockSpec((_BR, HID), lambda i: (i, 0)),
            pl.BlockSpec((_BR, 1), lambda i: (i, 0)),
            pl.BlockSpec((_BR, 1), lambda i: (i, 0)),
            pl.BlockSpec((_BR, 1), lambda i: (i, 0)),
            pl.BlockSpec((_BR, 1), lambda i: (i, 0)),
            pl.BlockSpec((_BR, HID), lambda i: (i, 0)),
            pl.BlockSpec((1, HID), lambda i: (0, 0)),
            pl.BlockSpec((HID, OUT), lambda i: (0, 0)),
            pl.BlockSpec((OUT, 1), lambda i: (0, 0)),
            pl.BlockSpec((OUT, 1), lambda i: (0, 0)),
        ],
        out_specs=[
            pl.BlockSpec((_BR, OUT), lambda i: (i, 0)),
            pl.BlockSpec((_BR, 1), lambda i: (i, 0)),
            pl.BlockSpec((_BR, 1), lambda i: (i, 0)),
        ],
        out_shape=[
            jax.ShapeDtypeStruct((N, OUT), jnp.float32),
            jax.ShapeDtypeStruct((N, 1), jnp.float32),
            jax.ShapeDtypeStruct((N, 1), jnp.float32),
        ],
    )(p0, p1, s0, s1, av, bv, h1, b1r, W2, a2s, a2d)


def _stage_c_body(p0_ref, p1_ref, s0_ref, s1_ref, av_ref, bv_ref, h2_ref,
                  b2_ref, y_ref):
    e = av_ref[...] + bv_ref[...]
    w = jnp.exp(jnp.maximum(e, 0.2 * e))
    num = p0_ref[...] + p1_ref[...] + w * h2_ref[...]
    den = s0_ref[...] + s1_ref[...] + w
    agg = num / den + b2_ref[...]
    m = jnp.max(agg, axis=1, keepdims=True)
    sh = agg - m
    y_ref[...] = sh - jnp.log(jnp.sum(jnp.exp(sh), axis=1, keepdims=True))


def _stage_c(p0, p1, s0, s1, av, bv, h2, b2r):
    return pl.pallas_call(
        _stage_c_body,
        grid=(_GRID,),
        in_specs=[
            pl.BlockSpec((_BR, OUT), lambda i: (i, 0)),
            pl.BlockSpec((_BR, OUT), lambda i: (i, 0)),
            pl.BlockSpec((_BR, 1), lambda i: (i, 0)),
            pl.BlockSpec((_BR, 1), lambda i: (i, 0)),
            pl.BlockSpec((_BR, 1), lambda i: (i, 0)),
            pl.BlockSpec((_BR, 1), lambda i: (i, 0)),
            pl.BlockSpec((_BR, OUT), lambda i: (i, 0)),
            pl.BlockSpec((1, OUT), lambda i: (0, 0)),
        ],
        out_specs=pl.BlockSpec((_BR, OUT), lambda i: (i, 0)),
        out_shape=jax.ShapeDtypeStruct((N, OUT), jnp.float32),
    )(p0, p1, s0, s1, av, bv, h2, b2r)


def _make_sc_agg(D):
    mesh = plsc.VectorSubcoreMesh(core_axis_name="c", subcore_axis_name="s",
                                  num_cores=NC, num_subcores=NS)

    @functools.partial(
        pl.kernel,
        out_type=(jax.ShapeDtypeStruct((NC * NPAD, D), jnp.float32),
                  jax.ShapeDtypeStruct((NC * NPAD,), jnp.float32)),
        mesh=mesh,
        scratch_types=[
            pltpu.VMEM((N,), jnp.float32),
            pltpu.VMEM((N,), jnp.float32),
            pltpu.VMEM((K,), jnp.int32),
            pltpu.VMEM((K,), jnp.int32),
            pltpu.VMEM((K,), jnp.float32),
            pltpu.VMEM((K, D), jnp.float32),
            pltpu.VMEM_SHARED((NPAD, D), jnp.float32),
            pltpu.VMEM_SHARED((NPAD,), jnp.float32),
            pltpu.SemaphoreType.DMA,
        ],
    )
    def sc_agg(src_hbm, dst_hbm, h_hbm, as_hbm, ad_hbm, zr_hbm, zs_hbm,
               out_hbm, sout_hbm,
               as_t, ad_t, src_v, dst_v, w_v, rows, out_acc, s_acc, sem):
        cid = lax.axis_index("c")
        sid = lax.axis_index("s")
        wid = sid * NC + cid

        r0 = sid * RPT
        pltpu.sync_copy(zr_hbm.at[pl.ds(r0, RPT)], out_acc.at[pl.ds(r0, RPT)])
        pltpu.sync_copy(zs_hbm.at[pl.ds(r0, RPT)], s_acc.at[pl.ds(r0, RPT)])
        pltpu.sync_copy(as_hbm, as_t)
        pltpu.sync_copy(ad_hbm, ad_t)
        plsc.subcore_barrier()

        base0 = wid * EW

        def chunk(c, carry):
            base = base0 + c * K
            pltpu.sync_copy(src_hbm.at[pl.ds(base, K)], src_v)
            pltpu.sync_copy(dst_hbm.at[pl.ds(base, K)], dst_v)
            for g in range(K // 16):
                sl = pl.ds(g * 16, 16)
                e = (plsc.load_gather(as_t, [src_v[sl]])
                     + plsc.load_gather(ad_t, [dst_v[sl]]))
                w_v[sl] = jnp.exp(jnp.maximum(e, 0.2 * e))
            pltpu.sync_copy(w_v, s_acc.at[dst_v], add=True)
            pltpu.async_copy(h_hbm.at[src_v], rows, sem).wait()

            def scale(k, carry2):
                wk = plsc.load_gather(w_v, [jnp.full((16,), k, jnp.int32)])
                row = rows.at[k]
                for j in range(D // 16):
                    cs = pl.ds(j * 16, 16)
                    row[cs] = row[cs] * wk
                return carry2

            lax.fori_loop(0, K, scale, 0)
            pltpu.sync_copy(rows, out_acc.at[dst_v], add=True)
            return carry

        lax.fori_loop(0, NCHUNK, chunk, 0)

        plsc.subcore_barrier()
        o0 = cid * NPAD + r0
        pltpu.sync_copy(out_acc.at[pl.ds(r0, RPT)], out_hbm.at[pl.ds(o0, RPT)])
        pltpu.sync_copy(s_acc.at[pl.ds(r0, RPT)], sout_hbm.at[pl.ds(o0, RPT)])

    return sc_agg


_sc_agg_hid = _make_sc_agg(HID)
_sc_agg_out = _make_sc_agg(OUT)


def kernel(x, edge_index, W1, att_src1, att_dst1, b1,
           W2, att_src2, att_dst2, b2):
    src = edge_index[0].astype(jnp.int32)
    dst = edge_index[1].astype(jnp.int32)

    h1, av1, bv1 = _stage_a(x, W1,
                            att_src1.reshape(HID, 1), att_dst1.reshape(HID, 1))

    zr1 = jnp.zeros((NPAD, HID), jnp.float32)
    zs = jnp.zeros((NPAD,), jnp.float32)
    out1, s1p = _sc_agg_hid(src, dst, h1,
                            av1.reshape(N), bv1.reshape(N), zr1, zs)
    p0 = out1[:N]
    p1 = out1[NPAD:NPAD + N]
    s0 = s1p[:N].reshape(N, 1)
    s1 = s1p[NPAD:NPAD + N].reshape(N, 1)

    h2, av2, bv2 = _stage_b(p0, p1, s0, s1, av1, bv1, h1,
                            b1.reshape(1, HID), W2,
                            att_src2.reshape(OUT, 1), att_dst2.reshape(OUT, 1))

    zr2 = jnp.zeros((NPAD, OUT), jnp.float32)
    out2, s2p = _sc_agg_out(src, dst, h2,
                            av2.reshape(N), bv2.reshape(N), zr2, zs)
    q0 = out2[:N]
    q1 = out2[NPAD:NPAD + N]
    t0 = s2p[:N].reshape(N, 1)
    t1 = s2p[NPAD:NPAD + N].reshape(N, 1)

    return _stage_c(q0, q1, t0, t1, av2, bv2, h2, b2.reshape(1, OUT))


# speedup vs baseline: 21.5959x; 21.5959x over previous
import functools

import jax
import jax.numpy as jnp
from jax import lax
from jax.experimental import pallas as pl
from jax.experimental.pallas import tpu as pltpu
from jax.experimental.pallas import tpu_sc as plsc

N = 10000
E = 320000
D_IN = 128
HID = 128
OUT = 64

NC = 2
NS = 16
NW = NC * NS
EW = E // NW
K = 80
NCHUNK = EW // K
RPT = 632
NPAD = RPT * NS

_BR = 1000
_GRID = N // _BR


def _dot(a, b):
    return jnp.dot(a, b, precision=lax.Precision.HIGHEST,
                   preferred_element_type=jnp.float32)


def _stage_a_body(x_ref, w_ref, at_s_ref, at_d_ref, h_ref, av_ref, bv_ref):
    h = _dot(x_ref[...], w_ref[...])
    h_ref[...] = h
    av_ref[...] = _dot(h, at_s_ref[...])
    bv_ref[...] = _dot(h, at_d_ref[...])


def _stage_a(x, W1, at_s, at_d):
    return pl.pallas_call(
        _stage_a_body,
        grid=(_GRID,),
        in_specs=[
            pl.BlockSpec((_BR, D_IN), lambda i: (i, 0)),
            pl.BlockSpec((D_IN, HID), lambda i: (0, 0)),
            pl.BlockSpec((HID, 1), lambda i: (0, 0)),
            pl.BlockSpec((HID, 1), lambda i: (0, 0)),
        ],
        out_specs=[
            pl.BlockSpec((_BR, HID), lambda i: (i, 0)),
            pl.BlockSpec((_BR, 1), lambda i: (i, 0)),
            pl.BlockSpec((_BR, 1), lambda i: (i, 0)),
        ],
        out_shape=[
            jax.ShapeDtypeStruct((N, HID), jnp.float32),
            jax.ShapeDtypeStruct((N, 1), jnp.float32),
            jax.ShapeDtypeStruct((N, 1), jnp.float32),
        ],
    )(x, W1, at_s, at_d)


def _stage_b_body(p0_ref, p1_ref, s0_ref, s1_ref, av_ref, bv_ref, h1_ref,
                  b1_ref, w2_ref, a2s_ref, a2d_ref, h2_ref, av2_ref, bv2_ref):
    e = av_ref[...] + bv_ref[...]
    w = jnp.exp(jnp.maximum(e, 0.2 * e))
    num = p0_ref[...] + p1_ref[...] + w * h1_ref[...]
    den = s0_ref[...] + s1_ref[...] + w
    agg = num / den + b1_ref[...]
    x2 = jnp.where(agg > 0, agg, jnp.exp(agg) - 1.0)
    h2 = _dot(x2, w2_ref[...])
    h2_ref[...] = jnp.concatenate(
        [h2, jnp.zeros((h2.shape[0], HID - OUT), jnp.float32)], axis=1)
    av2_ref[...] = _dot(h2, a2s_ref[...])
    bv2_ref[...] = _dot(h2, a2d_ref[...])


def _stage_b(p0, p1, s0, s1, av, bv, h1, b1r, W2, a2s, a2d):
    return pl.pallas_call(
        _stage_b_body,
        grid=(_GRID,),
        in_specs=[
            pl.BlockSpec((_BR, HID), lambda i: (i, 0)),
            pl.BlockSpec((_BR, HID), lambda i: (i, 0)),
            pl.BlockSpec((_BR, 1), lambda i: (i, 0)),
            pl.BlockSpec((_BR, 1), lambda i: (i, 0)),
            pl.BlockSpec((_BR, 1), lambda i: (i, 0)),
            pl.BlockSpec((_BR, 1), lambda i: (i, 0)),
            pl.BlockSpec((_BR, HID), lambda i: (i, 0)),
            pl.BlockSpec((1, HID), lambda i: (0, 0)),
            pl.BlockSpec((HID, OUT), lambda i: (0, 0)),
            pl.BlockSpec((OUT, 1), lambda i: (0, 0)),
            pl.BlockSpec((OUT, 1), lambda i: (0, 0)),
        ],
        out_specs=[
            pl.BlockSpec((_BR, HID), lambda i: (i, 0)),
            pl.BlockSpec((_BR, 1), lambda i: (i, 0)),
            pl.BlockSpec((_BR, 1), lambda i: (i, 0)),
        ],
        out_shape=[
            jax.ShapeDtypeStruct((N, HID), jnp.float32),
            jax.ShapeDtypeStruct((N, 1), jnp.float32),
            jax.ShapeDtypeStruct((N, 1), jnp.float32),
        ],
    )(p0, p1, s0, s1, av, bv, h1, b1r, W2, a2s, a2d)


def _stage_c_body(p0_ref, p1_ref, s0_ref, s1_ref, av_ref, bv_ref, h2_ref,
                  b2_ref, y_ref):
    e = av_ref[...] + bv_ref[...]
    w = jnp.exp(jnp.maximum(e, 0.2 * e))
    num = p0_ref[...] + p1_ref[...] + w * h2_ref[...]
    den = s0_ref[...] + s1_ref[...] + w
    agg = num / den + b2_ref[...]
    m = jnp.max(agg, axis=1, keepdims=True)
    sh = agg - m
    y_ref[...] = sh - jnp.log(jnp.sum(jnp.exp(sh), axis=1, keepdims=True))


def _stage_c(p0, p1, s0, s1, av, bv, h2, b2r):
    return pl.pallas_call(
        _stage_c_body,
        grid=(_GRID,),
        in_specs=[
            pl.BlockSpec((_BR, OUT), lambda i: (i, 0)),
            pl.BlockSpec((_BR, OUT), lambda i: (i, 0)),
            pl.BlockSpec((_BR, 1), lambda i: (i, 0)),
            pl.BlockSpec((_BR, 1), lambda i: (i, 0)),
            pl.BlockSpec((_BR, 1), lambda i: (i, 0)),
            pl.BlockSpec((_BR, 1), lambda i: (i, 0)),
            pl.BlockSpec((_BR, OUT), lambda i: (i, 0)),
            pl.BlockSpec((1, OUT), lambda i: (0, 0)),
        ],
        out_specs=pl.BlockSpec((_BR, OUT), lambda i: (i, 0)),
        out_shape=jax.ShapeDtypeStruct((N, OUT), jnp.float32),
    )(p0, p1, s0, s1, av, bv, h2, b2r)


def _make_sc_agg(d_scale):
    D = HID
    mesh = plsc.VectorSubcoreMesh(core_axis_name="c", subcore_axis_name="s",
                                  num_cores=NC, num_subcores=NS)

    @functools.partial(
        pl.kernel,
        out_type=(jax.ShapeDtypeStruct((NC * NPAD, D), jnp.float32),
                  jax.ShapeDtypeStruct((NC * NPAD,), jnp.float32)),
        mesh=mesh,
        compiler_params=pltpu.CompilerParams(needs_layout_passes=False),
        scratch_types=[
            pltpu.VMEM((N,), jnp.float32),
            pltpu.VMEM((N,), jnp.float32),
            pltpu.VMEM((K,), jnp.int32),
            pltpu.VMEM((K,), jnp.int32),
            pltpu.VMEM((K,), jnp.float32),
            pltpu.VMEM((K, D), jnp.float32),
            pltpu.VMEM((640,), jnp.float32),
            pltpu.VMEM_SHARED((NPAD, D), jnp.float32),
            pltpu.VMEM_SHARED((NPAD,), jnp.float32),
            pltpu.SemaphoreType.DMA,
        ],
    )
    def sc_agg(src_hbm, dst_hbm, h_hbm, as_hbm, ad_hbm,
               out_hbm, sout_hbm,
               as_t, ad_t, src_v, dst_v, w_v, rows, zbuf, out_acc, s_acc,
               sem):
        cid = lax.axis_index("c")
        sid = lax.axis_index("s")
        wid = sid * NC + cid

        zv = jnp.zeros((16,), jnp.float32)

        def zrow(k, carry):
            row = rows.at[k]
            for j in range(D // 16):
                row[pl.ds(j * 16, 16)] = zv
            return carry

        lax.fori_loop(0, K, zrow, 0)
        for g in range(640 // 16):
            zbuf[pl.ds(g * 16, 16)] = zv

        r0 = sid * RPT
        off = 0
        while off < RPT:
            sz = min(K, RPT - off)
            pltpu.sync_copy(rows.at[pl.ds(0, sz)],
                            out_acc.at[pl.ds(r0 + off, sz)])
            off += sz
        pltpu.sync_copy(zbuf.at[pl.ds(0, RPT)], s_acc.at[pl.ds(r0, RPT)])
        pltpu.sync_copy(as_hbm, as_t)
        pltpu.sync_copy(ad_hbm, ad_t)
        plsc.subcore_barrier()

        base0 = wid * EW

        def chunk(c, carry):
            base = base0 + c * K
            pltpu.sync_copy(src_hbm.at[pl.ds(base, K)], src_v)
            pltpu.sync_copy(dst_hbm.at[pl.ds(base, K)], dst_v)
            for g in range(K // 16):
                sl = pl.ds(g * 16, 16)
                e = (plsc.load_gather(as_t, [src_v[sl]])
                     + plsc.load_gather(ad_t, [dst_v[sl]]))
                w_v[sl] = jnp.exp(jnp.maximum(e, 0.2 * e))
            pltpu.sync_copy(w_v, s_acc.at[dst_v], add=True)
            pltpu.async_copy(h_hbm.at[src_v], rows, sem).wait()

            def scale(k, carry2):
                wk = plsc.load_gather(w_v, [jnp.full((16,), k, jnp.int32)])
                row = rows.at[k]
                for j in range(d_scale // 16):
                    cs = pl.ds(j * 16, 16)
                    row[cs] = row[cs] * wk
                return carry2

            lax.fori_loop(0, K, scale, 0)
            pltpu.sync_copy(rows, out_acc.at[dst_v], add=True)
            return carry

        lax.fori_loop(0, NCHUNK, chunk, 0)

        plsc.subcore_barrier()
        o0 = cid * NPAD + r0
        off = 0
        while off < RPT:
            sz = min(K, RPT - off)
            pltpu.sync_copy(out_acc.at[pl.ds(r0 + off, sz)],
                            rows.at[pl.ds(0, sz)])
            pltpu.sync_copy(rows.at[pl.ds(0, sz)],
                            out_hbm.at[pl.ds(o0 + off, sz)])
            off += sz
        pltpu.sync_copy(s_acc.at[pl.ds(r0, RPT)], zbuf.at[pl.ds(0, RPT)])
        pltpu.sync_copy(zbuf.at[pl.ds(0, RPT)], sout_hbm.at[pl.ds(o0, RPT)])

    return sc_agg


_sc_agg_hid = _make_sc_agg(HID)
_sc_agg_out = _make_sc_agg(OUT)


def kernel(x, edge_index, W1, att_src1, att_dst1, b1,
           W2, att_src2, att_dst2, b2):
    src = edge_index[0].astype(jnp.int32)
    dst = edge_index[1].astype(jnp.int32)

    h1, av1, bv1 = _stage_a(x, W1,
                            att_src1.reshape(HID, 1), att_dst1.reshape(HID, 1))

    out1, s1p = _sc_agg_hid(src, dst, h1, av1.reshape(N), bv1.reshape(N))
    p0 = out1[:N]
    p1 = out1[NPAD:NPAD + N]
    s0 = s1p[:N].reshape(N, 1)
    s1 = s1p[NPAD:NPAD + N].reshape(N, 1)

    h2p, av2, bv2 = _stage_b(p0, p1, s0, s1, av1, bv1, h1,
                             b1.reshape(1, HID), W2,
                             att_src2.reshape(OUT, 1), att_dst2.reshape(OUT, 1))

    out2, s2p = _sc_agg_out(src, dst, h2p, av2.reshape(N), bv2.reshape(N))
    q0 = out2[:N, :OUT]
    q1 = out2[NPAD:NPAD + N, :OUT]
    t0 = s2p[:N].reshape(N, 1)
    t1 = s2p[NPAD:NPAD + N].reshape(N, 1)

    return _stage_c(q0, q1, t0, t1, av2, bv2, h2p[:, :OUT],
                    b2.reshape(1, OUT))


# speedup vs baseline: 28.6674x; 1.3274x over previous
import functools

import jax
import jax.numpy as jnp
from jax import lax
from jax.experimental import pallas as pl
from jax.experimental.pallas import tpu as pltpu
from jax.experimental.pallas import tpu_sc as plsc

N = 10000
E = 320000
D_IN = 128
HID = 128
OUT = 64

NC = 2
NS = 16
NW = NC * NS
EW = E // NW
K = 80
NCHUNK = EW // K
RPT = 632
NPAD = RPT * NS

_BR = 1000
_GRID = N // _BR


def _dot(a, b):
    return jnp.dot(a, b, precision=lax.Precision.HIGHEST,
                   preferred_element_type=jnp.float32)


def _stage_a_body(x_ref, w_ref, at_s_ref, at_d_ref, h_ref, av_ref, bv_ref):
    h = _dot(x_ref[...], w_ref[...])
    h_ref[...] = h
    av_ref[...] = _dot(h, at_s_ref[...])
    bv_ref[...] = _dot(h, at_d_ref[...])


def _stage_a(x, W1, at_s, at_d):
    return pl.pallas_call(
        _stage_a_body,
        grid=(_GRID,),
        in_specs=[
            pl.BlockSpec((_BR, D_IN), lambda i: (i, 0)),
            pl.BlockSpec((D_IN, HID), lambda i: (0, 0)),
            pl.BlockSpec((HID, 1), lambda i: (0, 0)),
            pl.BlockSpec((HID, 1), lambda i: (0, 0)),
        ],
        out_specs=[
            pl.BlockSpec((_BR, HID), lambda i: (i, 0)),
            pl.BlockSpec((_BR, 1), lambda i: (i, 0)),
            pl.BlockSpec((_BR, 1), lambda i: (i, 0)),
        ],
        out_shape=[
            jax.ShapeDtypeStruct((N, HID), jnp.float32),
            jax.ShapeDtypeStruct((N, 1), jnp.float32),
            jax.ShapeDtypeStruct((N, 1), jnp.float32),
        ],
    )(x, W1, at_s, at_d)


def _stage_b_body(p0_ref, p1_ref, s0_ref, s1_ref, av_ref, bv_ref, h1_ref,
                  b1_ref, w2_ref, a2s_ref, a2d_ref, h2_ref, av2_ref, bv2_ref):
    e = av_ref[...] + bv_ref[...]
    w = jnp.exp(jnp.maximum(e, 0.2 * e))
    num = p0_ref[...] + p1_ref[...] + w * h1_ref[...]
    den = s0_ref[...] + s1_ref[...] + w
    agg = num / den + b1_ref[...]
    x2 = jnp.where(agg > 0, agg, jnp.exp(agg) - 1.0)
    h2 = _dot(x2, w2_ref[...])
    h2_ref[...] = jnp.concatenate(
        [h2, jnp.zeros((h2.shape[0], HID - OUT), jnp.float32)], axis=1)
    av2_ref[...] = _dot(h2, a2s_ref[...])
    bv2_ref[...] = _dot(h2, a2d_ref[...])


def _stage_b(p0, p1, s0, s1, av, bv, h1, b1r, W2, a2s, a2d):
    return pl.pallas_call(
        _stage_b_body,
        grid=(_GRID,),
        in_specs=[
            pl.BlockSpec((_BR, HID), lambda i: (i, 0)),
            pl.BlockSpec((_BR, HID), lambda i: (i, 0)),
            pl.BlockSpec((_BR, 1), lambda i: (i, 0)),
            pl.BlockSpec((_BR, 1), lambda i: (i, 0)),
            pl.BlockSpec((_BR, 1), lambda i: (i, 0)),
            pl.BlockSpec((_BR, 1), lambda i: (i, 0)),
            pl.BlockSpec((_BR, HID), lambda i: (i, 0)),
            pl.BlockSpec((1, HID), lambda i: (0, 0)),
            pl.BlockSpec((HID, OUT), lambda i: (0, 0)),
            pl.BlockSpec((OUT, 1), lambda i: (0, 0)),
            pl.BlockSpec((OUT, 1), lambda i: (0, 0)),
        ],
        out_specs=[
            pl.BlockSpec((_BR, HID), lambda i: (i, 0)),
            pl.BlockSpec((_BR, 1), lambda i: (i, 0)),
            pl.BlockSpec((_BR, 1), lambda i: (i, 0)),
        ],
        out_shape=[
            jax.ShapeDtypeStruct((N, HID), jnp.float32),
            jax.ShapeDtypeStruct((N, 1), jnp.float32),
            jax.ShapeDtypeStruct((N, 1), jnp.float32),
        ],
    )(p0, p1, s0, s1, av, bv, h1, b1r, W2, a2s, a2d)


def _stage_c_body(p0_ref, p1_ref, s0_ref, s1_ref, av_ref, bv_ref, h2_ref,
                  b2_ref, y_ref):
    e = av_ref[...] + bv_ref[...]
    w = jnp.exp(jnp.maximum(e, 0.2 * e))
    num = p0_ref[...] + p1_ref[...] + w * h2_ref[...]
    den = s0_ref[...] + s1_ref[...] + w
    agg = num / den + b2_ref[...]
    m = jnp.max(agg, axis=1, keepdims=True)
    sh = agg - m
    y_ref[...] = sh - jnp.log(jnp.sum(jnp.exp(sh), axis=1, keepdims=True))


def _stage_c(p0, p1, s0, s1, av, bv, h2, b2r):
    return pl.pallas_call(
        _stage_c_body,
        grid=(_GRID,),
        in_specs=[
            pl.BlockSpec((_BR, OUT), lambda i: (i, 0)),
            pl.BlockSpec((_BR, OUT), lambda i: (i, 0)),
            pl.BlockSpec((_BR, 1), lambda i: (i, 0)),
            pl.BlockSpec((_BR, 1), lambda i: (i, 0)),
            pl.BlockSpec((_BR, 1), lambda i: (i, 0)),
            pl.BlockSpec((_BR, 1), lambda i: (i, 0)),
            pl.BlockSpec((_BR, OUT), lambda i: (i, 0)),
            pl.BlockSpec((1, OUT), lambda i: (0, 0)),
        ],
        out_specs=pl.BlockSpec((_BR, OUT), lambda i: (i, 0)),
        out_shape=jax.ShapeDtypeStruct((N, OUT), jnp.float32),
    )(p0, p1, s0, s1, av, bv, h2, b2r)


def _make_sc_agg(d_scale):
    D = HID
    mesh = plsc.VectorSubcoreMesh(core_axis_name="c", subcore_axis_name="s",
                                  num_cores=NC, num_subcores=NS)

    @functools.partial(
        pl.kernel,
        out_type=(jax.ShapeDtypeStruct((NC * NPAD, D), jnp.float32),
                  jax.ShapeDtypeStruct((NC * NPAD,), jnp.float32)),
        mesh=mesh,
        compiler_params=pltpu.CompilerParams(needs_layout_passes=False),
        scratch_types=[
            pltpu.VMEM((N,), jnp.float32),
            pltpu.VMEM((N,), jnp.float32),
            pltpu.VMEM((K,), jnp.int32),
            pltpu.VMEM((K,), jnp.int32),
            pltpu.VMEM((K,), jnp.int32),
            pltpu.VMEM((K,), jnp.int32),
            pltpu.VMEM((K,), jnp.float32),
            pltpu.VMEM((K,), jnp.float32),
            pltpu.VMEM((K, D), jnp.float32),
            pltpu.VMEM((K, D), jnp.float32),
            pltpu.VMEM((640,), jnp.float32),
            pltpu.VMEM_SHARED((NPAD, D), jnp.float32),
            pltpu.VMEM_SHARED((NPAD,), jnp.float32),
            pltpu.SemaphoreType.DMA,
            pltpu.SemaphoreType.DMA,
        ],
    )
    def sc_agg(src_hbm, dst_hbm, h_hbm, as_hbm, ad_hbm,
               out_hbm, sout_hbm,
               as_t, ad_t, si0, si1, di0, di1, w0, w1, g0, g1, zbuf,
               out_acc, s_acc, sem_g0, sem_g1):
        cid = lax.axis_index("c")
        sid = lax.axis_index("s")
        wid = sid * NC + cid
        sidx = (si0, si1)
        didx = (di0, di1)
        wbuf = (w0, w1)
        gbufs = (g0, g1)
        sem_g = (sem_g0, sem_g1)

        zv = jnp.zeros((16,), jnp.float32)

        def zrow(k, carry):
            row = g0.at[k]
            for j in range(D // 16):
                row[pl.ds(j * 16, 16)] = zv
            return carry

        lax.fori_loop(0, K, zrow, 0)
        for g in range(640 // 16):
            zbuf[pl.ds(g * 16, 16)] = zv

        r0 = sid * RPT
        off = 0
        while off < RPT:
            sz = min(K, RPT - off)
            pltpu.sync_copy(g0.at[pl.ds(0, sz)],
                            out_acc.at[pl.ds(r0 + off, sz)])
            off += sz
        pltpu.sync_copy(zbuf.at[pl.ds(0, RPT)], s_acc.at[pl.ds(r0, RPT)])
        pltpu.sync_copy(as_hbm, as_t)
        pltpu.sync_copy(ad_hbm, ad_t)
        plsc.subcore_barrier()

        base0 = wid * EW

        def prep(c, b):
            pltpu.sync_copy(src_hbm.at[pl.ds(base0 + c * K, K)], sidx[b])
            pltpu.sync_copy(dst_hbm.at[pl.ds(base0 + c * K, K)], didx[b])
            for g in range(K // 16):
                sl = pl.ds(g * 16, 16)
                e = (plsc.load_gather(as_t, [sidx[b][sl]])
                     + plsc.load_gather(ad_t, [didx[b][sl]]))
                wbuf[b][sl] = jnp.exp(jnp.maximum(e, 0.2 * e))
            pltpu.sync_copy(wbuf[b], s_acc.at[didx[b]], add=True)
            pltpu.async_copy(h_hbm.at[sidx[b]], gbufs[b], sem_g[b])

        def step(c, b, look):
            pltpu.make_async_copy(h_hbm.at[sidx[b]], gbufs[b],
                                  sem_g[b]).wait()
            gb = gbufs[b]
            wb = wbuf[b]

            def scale(kk, carry):
                for u in range(2):
                    k = kk * 2 + u
                    wk = plsc.load_gather(
                        wb, [jnp.full((16,), k, jnp.int32)])
                    grow = gb.at[k]
                    for j in range(d_scale // 16):
                        cs = pl.ds(j * 16, 16)
                        grow[cs] = grow[cs] * wk
                return carry

            lax.fori_loop(0, K // 2, scale, 0)
            pltpu.sync_copy(gb, out_acc.at[didx[b]], add=True)
            if look:
                prep(c + 2, b)

        prep(0, 0)
        prep(1, 1)

        def pair(i, carry):
            step(i * 2, 0, True)
            step(i * 2 + 1, 1, True)
            return carry

        lax.fori_loop(0, NCHUNK // 2 - 1, pair, 0)
        step(NCHUNK - 3, 0, True)
        step(NCHUNK - 2, 1, False)
        step(NCHUNK - 1, 0, False)

        plsc.subcore_barrier()
        o0 = cid * NPAD + r0
        off = 0
        while off < RPT:
            sz = min(K, RPT - off)
            pltpu.sync_copy(out_acc.at[pl.ds(r0 + off, sz)],
                            g0.at[pl.ds(0, sz)])
            pltpu.sync_copy(g0.at[pl.ds(0, sz)],
                            out_hbm.at[pl.ds(o0 + off, sz)])
            off += sz
        pltpu.sync_copy(s_acc.at[pl.ds(r0, RPT)], zbuf.at[pl.ds(0, RPT)])
        pltpu.sync_copy(zbuf.at[pl.ds(0, RPT)], sout_hbm.at[pl.ds(o0, RPT)])

    return sc_agg


_sc_agg_hid = _make_sc_agg(HID)
_sc_agg_out = _sc_agg_hid


def kernel(x, edge_index, W1, att_src1, att_dst1, b1,
           W2, att_src2, att_dst2, b2):
    src = edge_index[0].astype(jnp.int32)
    dst = edge_index[1].astype(jnp.int32)

    h1, av1, bv1 = _stage_a(x, W1,
                            att_src1.reshape(HID, 1), att_dst1.reshape(HID, 1))

    out1, s1p = _sc_agg_hid(src, dst, h1, av1.reshape(N), bv1.reshape(N))
    p0 = out1[:N]
    p1 = out1[NPAD:NPAD + N]
    s0 = s1p[:N].reshape(N, 1)
    s1 = s1p[NPAD:NPAD + N].reshape(N, 1)

    h2p, av2, bv2 = _stage_b(p0, p1, s0, s1, av1, bv1, h1,
                             b1.reshape(1, HID), W2,
                             att_src2.reshape(OUT, 1), att_dst2.reshape(OUT, 1))

    out2, s2p = _sc_agg_out(src, dst, h2p, av2.reshape(N), bv2.reshape(N))
    q0 = out2[:N, :OUT]
    q1 = out2[NPAD:NPAD + N, :OUT]
    t0 = s2p[:N].reshape(N, 1)
    t1 = s2p[NPAD:NPAD + N].reshape(N, 1)

    return _stage_c(q0, q1, t0, t1, av2, bv2, h2p[:, :OUT],
                    b2.reshape(1, OUT))


# speedup vs baseline: 46.8488x; 1.6342x over previous
import functools

import jax
import jax.numpy as jnp
from jax import lax
from jax.experimental import pallas as pl
from jax.experimental.pallas import tpu as pltpu
from jax.experimental.pallas import tpu_sc as plsc

N = 10000
E = 320000
D_IN = 128
HID = 128
OUT = 64

NC = 2
NS = 16
NW = NC * NS
EW = E // NW
K = 80
NCHUNK = EW // K
RPT = 632
NPAD = RPT * NS

_BR = 1000
_GRID = N // _BR


def _dot(a, b):
    return jnp.dot(a, b, precision=lax.Precision.HIGHEST,
                   preferred_element_type=jnp.float32)


def _stage_a_body(x_ref, w_ref, at_s_ref, at_d_ref, h_ref, av_ref, bv_ref):
    h = _dot(x_ref[...], w_ref[...])
    h_ref[...] = h
    av_ref[...] = _dot(h, at_s_ref[...])
    bv_ref[...] = _dot(h, at_d_ref[...])


def _stage_a(x, W1, at_s, at_d):
    return pl.pallas_call(
        _stage_a_body,
        grid=(_GRID,),
        in_specs=[
            pl.BlockSpec((_BR, D_IN), lambda i: (i, 0)),
            pl.BlockSpec((D_IN, HID), lambda i: (0, 0)),
            pl.BlockSpec((HID, 1), lambda i: (0, 0)),
            pl.BlockSpec((HID, 1), lambda i: (0, 0)),
        ],
        out_specs=[
            pl.BlockSpec((_BR, HID), lambda i: (i, 0)),
            pl.BlockSpec((_BR, 1), lambda i: (i, 0)),
            pl.BlockSpec((_BR, 1), lambda i: (i, 0)),
        ],
        out_shape=[
            jax.ShapeDtypeStruct((N, HID), jnp.float32),
            jax.ShapeDtypeStruct((N, 1), jnp.float32),
            jax.ShapeDtypeStruct((N, 1), jnp.float32),
        ],
    )(x, W1, at_s, at_d)


def _stage_b_body(p0_ref, p1_ref, s0_ref, s1_ref, av_ref, bv_ref, h1_ref,
                  b1_ref, w2_ref, a2s_ref, a2d_ref, h2_ref, av2_ref, bv2_ref):
    e = av_ref[...] + bv_ref[...]
    w = jnp.exp(jnp.maximum(e, 0.2 * e))
    num = p0_ref[...] + p1_ref[...] + w * h1_ref[...]
    den = s0_ref[...] + s1_ref[...] + w
    agg = num / den + b1_ref[...]
    x2 = jnp.where(agg > 0, agg, jnp.exp(agg) - 1.0)
    h2 = _dot(x2, w2_ref[...])
    h2_ref[...] = jnp.concatenate(
        [h2, jnp.zeros((h2.shape[0], HID - OUT), jnp.float32)], axis=1)
    av2_ref[...] = _dot(h2, a2s_ref[...])
    bv2_ref[...] = _dot(h2, a2d_ref[...])


def _stage_b(p0, p1, s0, s1, av, bv, h1, b1r, W2, a2s, a2d):
    return pl.pallas_call(
        _stage_b_body,
        grid=(_GRID,),
        in_specs=[
            pl.BlockSpec((_BR, HID), lambda i: (i, 0)),
            pl.BlockSpec((_BR, HID), lambda i: (i, 0)),
            pl.BlockSpec((_BR, 1), lambda i: (i, 0)),
            pl.BlockSpec((_BR, 1), lambda i: (i, 0)),
            pl.BlockSpec((_BR, 1), lambda i: (i, 0)),
            pl.BlockSpec((_BR, 1), lambda i: (i, 0)),
            pl.BlockSpec((_BR, HID), lambda i: (i, 0)),
            pl.BlockSpec((1, HID), lambda i: (0, 0)),
            pl.BlockSpec((HID, OUT), lambda i: (0, 0)),
            pl.BlockSpec((OUT, 1), lambda i: (0, 0)),
            pl.BlockSpec((OUT, 1), lambda i: (0, 0)),
        ],
        out_specs=[
            pl.BlockSpec((_BR, HID), lambda i: (i, 0)),
            pl.BlockSpec((_BR, 1), lambda i: (i, 0)),
            pl.BlockSpec((_BR, 1), lambda i: (i, 0)),
        ],
        out_shape=[
            jax.ShapeDtypeStruct((N, HID), jnp.float32),
            jax.ShapeDtypeStruct((N, 1), jnp.float32),
            jax.ShapeDtypeStruct((N, 1), jnp.float32),
        ],
    )(p0, p1, s0, s1, av, bv, h1, b1r, W2, a2s, a2d)


def _stage_c_body(p0_ref, p1_ref, s0_ref, s1_ref, av_ref, bv_ref, h2_ref,
                  b2_ref, y_ref):
    e = av_ref[...] + bv_ref[...]
    w = jnp.exp(jnp.maximum(e, 0.2 * e))
    num = p0_ref[...] + p1_ref[...] + w * h2_ref[...]
    den = s0_ref[...] + s1_ref[...] + w
    agg = num / den + b2_ref[...]
    m = jnp.max(agg, axis=1, keepdims=True)
    sh = agg - m
    y_ref[...] = sh - jnp.log(jnp.sum(jnp.exp(sh), axis=1, keepdims=True))


def _stage_c(p0, p1, s0, s1, av, bv, h2, b2r):
    return pl.pallas_call(
        _stage_c_body,
        grid=(_GRID,),
        in_specs=[
            pl.BlockSpec((_BR, OUT), lambda i: (i, 0)),
            pl.BlockSpec((_BR, OUT), lambda i: (i, 0)),
            pl.BlockSpec((_BR, 1), lambda i: (i, 0)),
            pl.BlockSpec((_BR, 1), lambda i: (i, 0)),
            pl.BlockSpec((_BR, 1), lambda i: (i, 0)),
            pl.BlockSpec((_BR, 1), lambda i: (i, 0)),
            pl.BlockSpec((_BR, OUT), lambda i: (i, 0)),
            pl.BlockSpec((1, OUT), lambda i: (0, 0)),
        ],
        out_specs=pl.BlockSpec((_BR, OUT), lambda i: (i, 0)),
        out_shape=jax.ShapeDtypeStruct((N, OUT), jnp.float32),
    )(p0, p1, s0, s1, av, bv, h2, b2r)


def _make_sc_agg(d_scale):
    D = HID
    mesh = plsc.VectorSubcoreMesh(core_axis_name="c", subcore_axis_name="s",
                                  num_cores=NC, num_subcores=NS)

    @functools.partial(
        pl.kernel,
        out_type=(jax.ShapeDtypeStruct((NC * NPAD, D), jnp.float32),
                  jax.ShapeDtypeStruct((NC * NPAD,), jnp.float32)),
        mesh=mesh,
        compiler_params=pltpu.CompilerParams(needs_layout_passes=False),
        scratch_types=[
            pltpu.VMEM((K, D), jnp.float32),
            pltpu.VMEM((K, D), jnp.float32),
            pltpu.VMEM((K, D), jnp.float32),
            pltpu.VMEM((K, D), jnp.float32),
            pltpu.VMEM((K,), jnp.int32),
            pltpu.VMEM((K,), jnp.int32),
            pltpu.VMEM((K,), jnp.int32),
            pltpu.VMEM((K,), jnp.int32),
            pltpu.VMEM((K,), jnp.int32),
            pltpu.VMEM((K,), jnp.int32),
            pltpu.VMEM((K,), jnp.int32),
            pltpu.VMEM((K,), jnp.int32),
            pltpu.VMEM((K,), jnp.float32),
            pltpu.VMEM((K,), jnp.float32),
            pltpu.VMEM((K,), jnp.float32),
            pltpu.VMEM((K,), jnp.float32),
            pltpu.VMEM((K,), jnp.float32),
            pltpu.VMEM((640,), jnp.float32),
            pltpu.VMEM_SHARED((NPAD, D), jnp.float32),
            pltpu.VMEM_SHARED((NPAD,), jnp.float32),
        ] + [pltpu.SemaphoreType.DMA] * 14,
    )
    def sc_agg(src_hbm, dst_hbm, h_hbm, as_hbm, ad_hbm,
               out_hbm, sout_hbm,
               g0, g1, g2, g3, si0, si1, si2, si3, di0, di1, di2, di3,
               av0, av1, bv0, bv1, wb, zbuf,
               out_acc, s_acc,
               sg0, sg1, sg2, sg3, ss0, ss1, ss2, ss3,
               sia, sib, sic, sid_, sa0, sa1):
        cid = lax.axis_index("c")
        sid = lax.axis_index("s")
        wid = sid * NC + cid
        gb = (g0, g1, g2, g3)
        sidx = (si0, si1, si2, si3)
        didx = (di0, di1, di2, di3)
        asv = (av0, av1)
        adv = (bv0, bv1)
        sem_g = (sg0, sg1, sg2, sg3)
        sem_s = (ss0, ss1, ss2, ss3)
        sem_i = (sia, sib, sic, sid_)
        sem_a = (sa0, sa1)

        zv = jnp.zeros((16,), jnp.float32)

        def zrow(k, carry):
            row = g0.at[k]
            for j in range(D // 16):
                row[pl.ds(j * 16, 16)] = zv
            return carry

        lax.fori_loop(0, K, zrow, 0)
        for g in range(640 // 16):
            zbuf[pl.ds(g * 16, 16)] = zv

        r0 = sid * RPT
        off = 0
        while off < RPT:
            sz = min(K, RPT - off)
            pltpu.sync_copy(g0.at[pl.ds(0, sz)],
                            out_acc.at[pl.ds(r0 + off, sz)])
            off += sz
        pltpu.sync_copy(zbuf.at[pl.ds(0, RPT)], s_acc.at[pl.ds(r0, RPT)])
        plsc.subcore_barrier()

        base0 = wid * EW

        def idx_copies(c, j):
            return (pltpu.make_async_copy(
                        src_hbm.at[pl.ds(base0 + c * K, K)], sidx[j],
                        sem_i[j]),
                    pltpu.make_async_copy(
                        dst_hbm.at[pl.ds(base0 + c * K, K)], didx[j],
                        sem_i[j]))

        def fire_idx(c, j):
            for cp in idx_copies(c, j):
                cp.start()

        def wait_idx(c, j):
            for cp in idx_copies(c, j):
                cp.wait()

        def att_copies(j, p):
            return (pltpu.make_async_copy(as_hbm.at[sidx[j]], asv[p],
                                          sem_a[p]),
                    pltpu.make_async_copy(ad_hbm.at[didx[j]], adv[p],
                                          sem_a[p]))

        def gather_copy(j):
            return pltpu.make_async_copy(h_hbm.at[sidx[j]], gb[j], sem_g[j])

        def fire_scatter(j):
            pltpu.async_copy(gb[j], out_acc.at[didx[j]], sem_s[j], add=True)

        def wait_scatter(j):
            pltpu.make_async_copy(gb[j], out_acc.at[didx[j]],
                                  sem_s[j]).wait()

        def step(c, j, ws=True, pi=True, ps=True):
            j1 = (j + 1) % 4
            j2 = (j + 2) % 4
            p = j % 2
            p1 = j1 % 2
            if ps:
                wait_idx(c + 1, j1)
                for cp in att_copies(j1, p1):
                    cp.start()
                gather_copy(j1).start()
            if ws:
                wait_scatter(j2)
            if pi:
                fire_idx(c + 2, j2)
            for cp in att_copies(j, p):
                cp.wait()
            for g in range(K // 16):
                sl = pl.ds(g * 16, 16)
                e = asv[p][sl] + adv[p][sl]
                wb[sl] = jnp.exp(jnp.maximum(e, 0.2 * e))
            pltpu.sync_copy(wb, s_acc.at[didx[j]], add=True)
            gather_copy(j).wait()
            gbj = gb[j]

            def scale(kk, carry):
                for u in range(2):
                    k = kk * 2 + u
                    wk = plsc.load_gather(
                        wb, [jnp.full((16,), k, jnp.int32)])
                    grow = gbj.at[k]
                    for jj in range(d_scale // 16):
                        cs = pl.ds(jj * 16, 16)
                        grow[cs] = grow[cs] * wk
                return carry

            lax.fori_loop(0, K // 2, scale, 0)
            fire_scatter(j)

        fire_idx(0, 0)
        fire_idx(1, 1)
        wait_idx(0, 0)
        for cp in att_copies(0, 0):
            cp.start()
        gather_copy(0).start()

        step(0, 0, ws=False)
        step(1, 1, ws=False)
        step(2, 2)
        step(3, 3)

        def quad(i, carry):
            c = i * 4
            step(c, 0)
            step(c + 1, 1)
            step(c + 2, 2)
            step(c + 3, 3)
            return carry

        lax.fori_loop(1, (NCHUNK - 1) // 4, quad, 0)
        step(NCHUNK - 1, 0, pi=False, ps=False)
        wait_scatter(3)
        wait_scatter(0)
        wait_idx(NCHUNK, 1)

        plsc.subcore_barrier()
        o0 = cid * NPAD + r0
        off = 0
        while off < RPT:
            sz = min(K, RPT - off)
            pltpu.sync_copy(out_acc.at[pl.ds(r0 + off, sz)],
                            g0.at[pl.ds(0, sz)])
            pltpu.sync_copy(g0.at[pl.ds(0, sz)],
                            out_hbm.at[pl.ds(o0 + off, sz)])
            off += sz
        pltpu.sync_copy(s_acc.at[pl.ds(r0, RPT)], zbuf.at[pl.ds(0, RPT)])
        pltpu.sync_copy(zbuf.at[pl.ds(0, RPT)], sout_hbm.at[pl.ds(o0, RPT)])

    return sc_agg


_sc_agg_hid = _make_sc_agg(HID)
_sc_agg_out = _sc_agg_hid


def kernel(x, edge_index, W1, att_src1, att_dst1, b1,
           W2, att_src2, att_dst2, b2):
    zk = jnp.zeros((K,), jnp.int32)
    zn = jnp.zeros((NPAD - N,), jnp.float32)
    src = jnp.concatenate([edge_index[0].astype(jnp.int32), zk])
    dst = jnp.concatenate([edge_index[1].astype(jnp.int32), zk])

    h1, av1, bv1 = _stage_a(x, W1,
                            att_src1.reshape(HID, 1), att_dst1.reshape(HID, 1))

    out1, s1p = _sc_agg_hid(src, dst, h1,
                            jnp.concatenate([av1.reshape(N), zn]),
                            jnp.concatenate([bv1.reshape(N), zn]))
    p0 = out1[:N]
    p1 = out1[NPAD:NPAD + N]
    s0 = s1p[:N].reshape(N, 1)
    s1 = s1p[NPAD:NPAD + N].reshape(N, 1)

    h2p, av2, bv2 = _stage_b(p0, p1, s0, s1, av1, bv1, h1,
                             b1.reshape(1, HID), W2,
                             att_src2.reshape(OUT, 1), att_dst2.reshape(OUT, 1))

    out2, s2p = _sc_agg_out(src, dst, h2p,
                            jnp.concatenate([av2.reshape(N), zn]),
                            jnp.concatenate([bv2.reshape(N), zn]))
    q0 = out2[:N, :OUT]
    q1 = out2[NPAD:NPAD + N, :OUT]
    t0 = s2p[:N].reshape(N, 1)
    t1 = s2p[NPAD:NPAD + N].reshape(N, 1)

    return _stage_c(q0, q1, t0, t1, av2, bv2, h2p[:, :OUT],
                    b2.reshape(1, OUT))


# speedup vs baseline: 48.0740x; 1.0262x over previous
import functools

import jax
import jax.numpy as jnp
from jax import lax
from jax.experimental import pallas as pl
from jax.experimental.pallas import tpu as pltpu
from jax.experimental.pallas import tpu_sc as plsc

N = 10000
E = 320000
D_IN = 128
HID = 128
OUT = 64

NC = 2
NS = 16
NW = NC * NS
EW = E // NW
K = 80
NCHUNK = EW // K
RPT = 632
NPAD = RPT * NS

_BR = 1000
_GRID = N // _BR


def _dot(a, b):
    return jnp.dot(a, b, precision=lax.Precision.HIGHEST,
                   preferred_element_type=jnp.float32)


def _stage_a_body(x_ref, w_ref, at_s_ref, at_d_ref, h_ref, av_ref, bv_ref):
    h = _dot(x_ref[...], w_ref[...])
    h_ref[...] = h
    av_ref[...] = _dot(h, at_s_ref[...])
    bv_ref[...] = _dot(h, at_d_ref[...])


def _stage_a(x, W1, at_s, at_d):
    return pl.pallas_call(
        _stage_a_body,
        grid=(_GRID,),
        in_specs=[
            pl.BlockSpec((_BR, D_IN), lambda i: (i, 0)),
            pl.BlockSpec((D_IN, HID), lambda i: (0, 0)),
            pl.BlockSpec((HID, 1), lambda i: (0, 0)),
            pl.BlockSpec((HID, 1), lambda i: (0, 0)),
        ],
        out_specs=[
            pl.BlockSpec((_BR, HID), lambda i: (i, 0)),
            pl.BlockSpec((_BR, 1), lambda i: (i, 0)),
            pl.BlockSpec((_BR, 1), lambda i: (i, 0)),
        ],
        out_shape=[
            jax.ShapeDtypeStruct((N, HID), jnp.float32),
            jax.ShapeDtypeStruct((N, 1), jnp.float32),
            jax.ShapeDtypeStruct((N, 1), jnp.float32),
        ],
    )(x, W1, at_s, at_d)


def _stage_b_body(p0_ref, p1_ref, s0_ref, s1_ref, av_ref, bv_ref, h1_ref,
                  b1_ref, w2_ref, a2s_ref, a2d_ref, h2_ref, av2_ref, bv2_ref):
    e = av_ref[...] + bv_ref[...]
    w = jnp.exp(jnp.maximum(e, 0.2 * e))
    num = p0_ref[...] + p1_ref[...] + w * h1_ref[...]
    den = s0_ref[...] + s1_ref[...] + w
    agg = num / den + b1_ref[...]
    x2 = jnp.where(agg > 0, agg, jnp.exp(agg) - 1.0)
    h2 = _dot(x2, w2_ref[...])
    h2_ref[...] = jnp.concatenate(
        [h2, jnp.zeros((h2.shape[0], HID - OUT), jnp.float32)], axis=1)
    av2_ref[...] = _dot(h2, a2s_ref[...])
    bv2_ref[...] = _dot(h2, a2d_ref[...])


def _stage_b(p0, p1, s0, s1, av, bv, h1, b1r, W2, a2s, a2d):
    return pl.pallas_call(
        _stage_b_body,
        grid=(_GRID,),
        in_specs=[
            pl.BlockSpec((_BR, HID), lambda i: (i, 0)),
            pl.BlockSpec((_BR, HID), lambda i: (i, 0)),
            pl.BlockSpec((_BR, 1), lambda i: (i, 0)),
            pl.BlockSpec((_BR, 1), lambda i: (i, 0)),
            pl.BlockSpec((_BR, 1), lambda i: (i, 0)),
            pl.BlockSpec((_BR, 1), lambda i: (i, 0)),
            pl.BlockSpec((_BR, HID), lambda i: (i, 0)),
            pl.BlockSpec((1, HID), lambda i: (0, 0)),
            pl.BlockSpec((HID, OUT), lambda i: (0, 0)),
            pl.BlockSpec((OUT, 1), lambda i: (0, 0)),
            pl.BlockSpec((OUT, 1), lambda i: (0, 0)),
        ],
        out_specs=[
            pl.BlockSpec((_BR, HID), lambda i: (i, 0)),
            pl.BlockSpec((_BR, 1), lambda i: (i, 0)),
            pl.BlockSpec((_BR, 1), lambda i: (i, 0)),
        ],
        out_shape=[
            jax.ShapeDtypeStruct((N, HID), jnp.float32),
            jax.ShapeDtypeStruct((N, 1), jnp.float32),
            jax.ShapeDtypeStruct((N, 1), jnp.float32),
        ],
    )(p0, p1, s0, s1, av, bv, h1, b1r, W2, a2s, a2d)


def _stage_c_body(p0_ref, p1_ref, s0_ref, s1_ref, av_ref, bv_ref, h2_ref,
                  b2_ref, y_ref):
    e = av_ref[...] + bv_ref[...]
    w = jnp.exp(jnp.maximum(e, 0.2 * e))
    num = p0_ref[...] + p1_ref[...] + w * h2_ref[...]
    den = s0_ref[...] + s1_ref[...] + w
    agg = num / den + b2_ref[...]
    m = jnp.max(agg, axis=1, keepdims=True)
    sh = agg - m
    y_ref[...] = sh - jnp.log(jnp.sum(jnp.exp(sh), axis=1, keepdims=True))


def _stage_c(p0, p1, s0, s1, av, bv, h2, b2r):
    return pl.pallas_call(
        _stage_c_body,
        grid=(_GRID,),
        in_specs=[
            pl.BlockSpec((_BR, OUT), lambda i: (i, 0)),
            pl.BlockSpec((_BR, OUT), lambda i: (i, 0)),
            pl.BlockSpec((_BR, 1), lambda i: (i, 0)),
            pl.BlockSpec((_BR, 1), lambda i: (i, 0)),
            pl.BlockSpec((_BR, 1), lambda i: (i, 0)),
            pl.BlockSpec((_BR, 1), lambda i: (i, 0)),
            pl.BlockSpec((_BR, OUT), lambda i: (i, 0)),
            pl.BlockSpec((1, OUT), lambda i: (0, 0)),
        ],
        out_specs=pl.BlockSpec((_BR, OUT), lambda i: (i, 0)),
        out_shape=jax.ShapeDtypeStruct((N, OUT), jnp.float32),
    )(p0, p1, s0, s1, av, bv, h2, b2r)


def _make_sc_agg(d_scale):
    D = HID
    mesh = plsc.VectorSubcoreMesh(core_axis_name="c", subcore_axis_name="s",
                                  num_cores=NC, num_subcores=NS)

    @functools.partial(
        pl.kernel,
        out_type=(jax.ShapeDtypeStruct((NC * NPAD, D), jnp.float32),
                  jax.ShapeDtypeStruct((NC * NPAD,), jnp.float32)),
        mesh=mesh,
        compiler_params=pltpu.CompilerParams(needs_layout_passes=False),
        scratch_types=[
            pltpu.VMEM((K, D), jnp.float32),
            pltpu.VMEM((K, D), jnp.float32),
            pltpu.VMEM((K, D), jnp.float32),
            pltpu.VMEM((K, D), jnp.float32),
            pltpu.VMEM((K,), jnp.int32),
            pltpu.VMEM((K,), jnp.int32),
            pltpu.VMEM((K,), jnp.int32),
            pltpu.VMEM((K,), jnp.int32),
            pltpu.VMEM((K,), jnp.int32),
            pltpu.VMEM((K,), jnp.int32),
            pltpu.VMEM((K,), jnp.int32),
            pltpu.VMEM((K,), jnp.int32),
            pltpu.VMEM((K,), jnp.float32),
            pltpu.VMEM((K,), jnp.float32),
            pltpu.VMEM((K,), jnp.float32),
            pltpu.VMEM((K,), jnp.float32),
            pltpu.VMEM((K,), jnp.float32),
            pltpu.VMEM((640,), jnp.float32),
            pltpu.VMEM_SHARED((NPAD, D), jnp.float32),
            pltpu.VMEM_SHARED((NPAD,), jnp.float32),
        ] + [pltpu.SemaphoreType.DMA] * 14,
    )
    def sc_agg(src_hbm, dst_hbm, h_hbm, as_hbm, ad_hbm,
               out_hbm, sout_hbm,
               g0, g1, g2, g3, si0, si1, si2, si3, di0, di1, di2, di3,
               av0, av1, bv0, bv1, wb, zbuf,
               out_acc, s_acc,
               sg0, sg1, sg2, sg3, ss0, ss1, ss2, ss3,
               sia, sib, sic, sid_, sa0, sa1):
        cid = lax.axis_index("c")
        sid = lax.axis_index("s")
        wid = sid * NC + cid
        gb = (g0, g1, g2, g3)
        sidx = (si0, si1, si2, si3)
        didx = (di0, di1, di2, di3)
        asv = (av0, av1)
        adv = (bv0, bv1)
        sem_g = (sg0, sg1, sg2, sg3)
        sem_s = (ss0, ss1, ss2, ss3)
        sem_i = (sia, sib, sic, sid_)
        sem_a = (sa0, sa1)

        zv = jnp.zeros((16,), jnp.float32)

        def zrow(k, carry):
            row = g0.at[k]
            for j in range(D // 16):
                row[pl.ds(j * 16, 16)] = zv
            return carry

        lax.fori_loop(0, K, zrow, 0)
        for g in range(640 // 16):
            zbuf[pl.ds(g * 16, 16)] = zv

        r0 = sid * RPT
        off = 0
        while off < RPT:
            sz = min(K, RPT - off)
            pltpu.sync_copy(g0.at[pl.ds(0, sz)],
                            out_acc.at[pl.ds(r0 + off, sz)])
            off += sz
        pltpu.sync_copy(zbuf.at[pl.ds(0, RPT)], s_acc.at[pl.ds(r0, RPT)])
        plsc.subcore_barrier()

        base0 = wid * EW

        def idx_copies(c, j):
            return (pltpu.make_async_copy(
                        src_hbm.at[pl.ds(base0 + c * K, K)], sidx[j],
                        sem_i[j]),
                    pltpu.make_async_copy(
                        dst_hbm.at[pl.ds(base0 + c * K, K)], didx[j],
                        sem_i[j]))

        def fire_idx(c, j):
            for cp in idx_copies(c, j):
                cp.start()

        def wait_idx(c, j):
            for cp in idx_copies(c, j):
                cp.wait()

        def att_copies(j, p):
            return (pltpu.make_async_copy(as_hbm.at[sidx[j]], asv[p],
                                          sem_a[p]),
                    pltpu.make_async_copy(ad_hbm.at[didx[j]], adv[p],
                                          sem_a[p]))

        def gather_copy(j):
            return pltpu.make_async_copy(h_hbm.at[sidx[j]], gb[j], sem_g[j])

        def fire_scatter(j):
            pltpu.async_copy(gb[j], out_acc.at[didx[j]], sem_s[j], add=True)

        def wait_scatter(j):
            pltpu.make_async_copy(gb[j], out_acc.at[didx[j]],
                                  sem_s[j]).wait()

        def step(c, j, ws=True, pi=True, ps=True):
            j1 = (j + 1) % 4
            j2 = (j + 2) % 4
            p = j % 2
            p1 = j1 % 2
            if ps:
                wait_idx(c + 1, j1)
                for cp in att_copies(j1, p1):
                    cp.start()
                gather_copy(j1).start()
            if ws:
                wait_scatter(j2)
            if pi:
                fire_idx(c + 2, j2)
            for cp in att_copies(j, p):
                cp.wait()
            for g in range(K // 16):
                sl = pl.ds(g * 16, 16)
                e = asv[p][sl] + adv[p][sl]
                wb[sl] = jnp.exp(jnp.maximum(e, 0.2 * e))
            pltpu.sync_copy(wb, s_acc.at[didx[j]], add=True)
            gather_copy(j).wait()
            gbj = gb[j]

            def scale(kk, carry):
                for u in range(2):
                    k = kk * 2 + u
                    wk = plsc.load_gather(
                        wb, [jnp.full((16,), k, jnp.int32)])
                    grow = gbj.at[k]
                    for jj in range(d_scale // 16):
                        cs = pl.ds(jj * 16, 16)
                        grow[cs] = grow[cs] * wk
                return carry

            lax.fori_loop(0, K // 2, scale, 0)
            fire_scatter(j)

        fire_idx(0, 0)
        fire_idx(1, 1)
        wait_idx(0, 0)
        for cp in att_copies(0, 0):
            cp.start()
        gather_copy(0).start()

        step(0, 0, ws=False)
        step(1, 1, ws=False)
        step(2, 2)
        step(3, 3)

        def quad(i, carry):
            c = i * 4
            step(c, 0)
            step(c + 1, 1)
            step(c + 2, 2)
            step(c + 3, 3)
            return carry

        lax.fori_loop(1, (NCHUNK - 1) // 4, quad, 0)
        step(NCHUNK - 1, 0, pi=False, ps=False)
        wait_scatter(3)
        wait_scatter(0)
        wait_idx(NCHUNK, 1)

        plsc.subcore_barrier()
        o0 = cid * NPAD + r0
        off = 0
        while off < RPT:
            sz = min(K, RPT - off)
            pltpu.sync_copy(out_acc.at[pl.ds(r0 + off, sz)],
                            g0.at[pl.ds(0, sz)])
            pltpu.sync_copy(g0.at[pl.ds(0, sz)],
                            out_hbm.at[pl.ds(o0 + off, sz)])
            off += sz
        pltpu.sync_copy(s_acc.at[pl.ds(r0, RPT)], zbuf.at[pl.ds(0, RPT)])
        pltpu.sync_copy(zbuf.at[pl.ds(0, RPT)], sout_hbm.at[pl.ds(o0, RPT)])

    return sc_agg


_sc_agg_hid = _make_sc_agg(HID)
_sc_agg_out = _make_sc_agg(OUT)


def kernel(x, edge_index, W1, att_src1, att_dst1, b1,
           W2, att_src2, att_dst2, b2):
    zk = jnp.zeros((K,), jnp.int32)
    zn = jnp.zeros((NPAD - N,), jnp.float32)
    src = jnp.concatenate([edge_index[0].astype(jnp.int32), zk])
    dst = jnp.concatenate([edge_index[1].astype(jnp.int32), zk])

    h1, av1, bv1 = _stage_a(x, W1,
                            att_src1.reshape(HID, 1), att_dst1.reshape(HID, 1))

    out1, s1p = _sc_agg_hid(src, dst, h1,
                            jnp.concatenate([av1.reshape(N), zn]),
                            jnp.concatenate([bv1.reshape(N), zn]))
    p0 = out1[:N]
    p1 = out1[NPAD:NPAD + N]
    s0 = s1p[:N].reshape(N, 1)
    s1 = s1p[NPAD:NPAD + N].reshape(N, 1)

    h2p, av2, bv2 = _stage_b(p0, p1, s0, s1, av1, bv1, h1,
                             b1.reshape(1, HID), W2,
                             att_src2.reshape(OUT, 1), att_dst2.reshape(OUT, 1))

    out2, s2p = _sc_agg_out(src, dst, h2p,
                            jnp.concatenate([av2.reshape(N), zn]),
                            jnp.concatenate([bv2.reshape(N), zn]))
    q0 = out2[:N, :OUT]
    q1 = out2[NPAD:NPAD + N, :OUT]
    t0 = s2p[:N].reshape(N, 1)
    t1 = s2p[NPAD:NPAD + N].reshape(N, 1)

    return _stage_c(q0, q1, t0, t1, av2, bv2, h2p[:, :OUT],
                    b2.reshape(1, OUT))


# speedup vs baseline: 52.8564x; 1.0995x over previous
import functools

import jax
import jax.numpy as jnp
from jax import lax
from jax.experimental import pallas as pl
from jax.experimental.pallas import tpu as pltpu
from jax.experimental.pallas import tpu_sc as plsc

N = 10000
E = 320000
D_IN = 128
HID = 128
OUT = 64

NC = 2
NS = 16
NW = NC * NS
EW = E // NW
K = 80
NCHUNK = EW // K
RPT = 632
NPAD = RPT * NS

_BR = 10000
_GRID = N // _BR


def _dot(a, b):
    return jnp.dot(a, b, preferred_element_type=jnp.float32)


def _stage_a_body(x_ref, w_ref, at_s_ref, at_d_ref, h_ref, av_ref, bv_ref):
    h = _dot(x_ref[...], w_ref[...])
    h_ref[...] = h
    av_ref[...] = _dot(h, at_s_ref[...])
    bv_ref[...] = _dot(h, at_d_ref[...])


def _stage_a(x, W1, at_s, at_d):
    return pl.pallas_call(
        _stage_a_body,
        grid=(_GRID,),
        in_specs=[
            pl.BlockSpec((_BR, D_IN), lambda i: (i, 0)),
            pl.BlockSpec((D_IN, HID), lambda i: (0, 0)),
            pl.BlockSpec((HID, 1), lambda i: (0, 0)),
            pl.BlockSpec((HID, 1), lambda i: (0, 0)),
        ],
        out_specs=[
            pl.BlockSpec((_BR, HID), lambda i: (i, 0)),
            pl.BlockSpec((_BR, 1), lambda i: (i, 0)),
            pl.BlockSpec((_BR, 1), lambda i: (i, 0)),
        ],
        out_shape=[
            jax.ShapeDtypeStruct((N, HID), jnp.float32),
            jax.ShapeDtypeStruct((N, 1), jnp.float32),
            jax.ShapeDtypeStruct((N, 1), jnp.float32),
        ],
    )(x, W1, at_s, at_d)


def _stage_b_body(p0_ref, p1_ref, s0_ref, s1_ref, av_ref, bv_ref, h1_ref,
                  b1_ref, w2_ref, a2s_ref, a2d_ref, h2_ref, av2_ref, bv2_ref):
    e = av_ref[...] + bv_ref[...]
    w = jnp.exp(jnp.maximum(e, 0.2 * e))
    num = p0_ref[...] + p1_ref[...] + w * h1_ref[...]
    den = s0_ref[...] + s1_ref[...] + w
    agg = num / den + b1_ref[...]
    x2 = jnp.where(agg > 0, agg, jnp.exp(agg) - 1.0)
    h2 = _dot(x2, w2_ref[...])
    h2_ref[...] = jnp.concatenate(
        [h2, jnp.zeros((h2.shape[0], HID - OUT), jnp.float32)], axis=1)
    av2_ref[...] = _dot(h2, a2s_ref[...])
    bv2_ref[...] = _dot(h2, a2d_ref[...])


def _stage_b(p0, p1, s0, s1, av, bv, h1, b1r, W2, a2s, a2d):
    return pl.pallas_call(
        _stage_b_body,
        grid=(_GRID,),
        in_specs=[
            pl.BlockSpec((_BR, HID), lambda i: (i, 0)),
            pl.BlockSpec((_BR, HID), lambda i: (i, 0)),
            pl.BlockSpec((_BR, 1), lambda i: (i, 0)),
            pl.BlockSpec((_BR, 1), lambda i: (i, 0)),
            pl.BlockSpec((_BR, 1), lambda i: (i, 0)),
            pl.BlockSpec((_BR, 1), lambda i: (i, 0)),
            pl.BlockSpec((_BR, HID), lambda i: (i, 0)),
            pl.BlockSpec((1, HID), lambda i: (0, 0)),
            pl.BlockSpec((HID, OUT), lambda i: (0, 0)),
            pl.BlockSpec((OUT, 1), lambda i: (0, 0)),
            pl.BlockSpec((OUT, 1), lambda i: (0, 0)),
        ],
        out_specs=[
            pl.BlockSpec((_BR, HID), lambda i: (i, 0)),
            pl.BlockSpec((_BR, 1), lambda i: (i, 0)),
            pl.BlockSpec((_BR, 1), lambda i: (i, 0)),
        ],
        out_shape=[
            jax.ShapeDtypeStruct((N, HID), jnp.float32),
            jax.ShapeDtypeStruct((N, 1), jnp.float32),
            jax.ShapeDtypeStruct((N, 1), jnp.float32),
        ],
    )(p0, p1, s0, s1, av, bv, h1, b1r, W2, a2s, a2d)


def _stage_c_body(p0_ref, p1_ref, s0_ref, s1_ref, av_ref, bv_ref, h2_ref,
                  b2_ref, y_ref):
    e = av_ref[...] + bv_ref[...]
    w = jnp.exp(jnp.maximum(e, 0.2 * e))
    num = p0_ref[...] + p1_ref[...] + w * h2_ref[...]
    den = s0_ref[...] + s1_ref[...] + w
    agg = num / den + b2_ref[...]
    m = jnp.max(agg, axis=1, keepdims=True)
    sh = agg - m
    y_ref[...] = sh - jnp.log(jnp.sum(jnp.exp(sh), axis=1, keepdims=True))


def _stage_c(p0, p1, s0, s1, av, bv, h2, b2r):
    return pl.pallas_call(
        _stage_c_body,
        grid=(_GRID,),
        in_specs=[
            pl.BlockSpec((_BR, OUT), lambda i: (i, 0)),
            pl.BlockSpec((_BR, OUT), lambda i: (i, 0)),
            pl.BlockSpec((_BR, 1), lambda i: (i, 0)),
            pl.BlockSpec((_BR, 1), lambda i: (i, 0)),
            pl.BlockSpec((_BR, 1), lambda i: (i, 0)),
            pl.BlockSpec((_BR, 1), lambda i: (i, 0)),
            pl.BlockSpec((_BR, OUT), lambda i: (i, 0)),
            pl.BlockSpec((1, OUT), lambda i: (0, 0)),
        ],
        out_specs=pl.BlockSpec((_BR, OUT), lambda i: (i, 0)),
        out_shape=jax.ShapeDtypeStruct((N, OUT), jnp.float32),
    )(p0, p1, s0, s1, av, bv, h2, b2r)


def _make_sc_agg(d_scale):
    D = HID
    mesh = plsc.VectorSubcoreMesh(core_axis_name="c", subcore_axis_name="s",
                                  num_cores=NC, num_subcores=NS)

    @functools.partial(
        pl.kernel,
        out_type=(jax.ShapeDtypeStruct((NC * NPAD, D), jnp.float32),
                  jax.ShapeDtypeStruct((NC * NPAD,), jnp.float32)),
        mesh=mesh,
        compiler_params=pltpu.CompilerParams(needs_layout_passes=False),
        scratch_types=[
            pltpu.VMEM((K, D), jnp.float32),
            pltpu.VMEM((K, D), jnp.float32),
            pltpu.VMEM((K, D), jnp.float32),
            pltpu.VMEM((K, D), jnp.float32),
            pltpu.VMEM((K,), jnp.int32),
            pltpu.VMEM((K,), jnp.int32),
            pltpu.VMEM((K,), jnp.int32),
            pltpu.VMEM((K,), jnp.int32),
            pltpu.VMEM((K,), jnp.int32),
            pltpu.VMEM((K,), jnp.int32),
            pltpu.VMEM((K,), jnp.int32),
            pltpu.VMEM((K,), jnp.int32),
            pltpu.VMEM((K,), jnp.float32),
            pltpu.VMEM((K,), jnp.float32),
            pltpu.VMEM((K,), jnp.float32),
            pltpu.VMEM((K,), jnp.float32),
            pltpu.VMEM((K,), jnp.float32),
            pltpu.VMEM((640,), jnp.float32),
            pltpu.VMEM_SHARED((NPAD, D), jnp.float32),
            pltpu.VMEM_SHARED((NPAD,), jnp.float32),
        ] + [pltpu.SemaphoreType.DMA] * 14,
    )
    def sc_agg(src_hbm, dst_hbm, h_hbm, as_hbm, ad_hbm,
               out_hbm, sout_hbm,
               g0, g1, g2, g3, si0, si1, si2, si3, di0, di1, di2, di3,
               av0, av1, bv0, bv1, wb, zbuf,
               out_acc, s_acc,
               sg0, sg1, sg2, sg3, ss0, ss1, ss2, ss3,
               sia, sib, sic, sid_, sa0, sa1):
        cid = lax.axis_index("c")
        sid = lax.axis_index("s")
        wid = sid * NC + cid
        gb = (g0, g1, g2, g3)
        sidx = (si0, si1, si2, si3)
        didx = (di0, di1, di2, di3)
        asv = (av0, av1)
        adv = (bv0, bv1)
        sem_g = (sg0, sg1, sg2, sg3)
        sem_s = (ss0, ss1, ss2, ss3)
        sem_i = (sia, sib, sic, sid_)
        sem_a = (sa0, sa1)

        zv = jnp.zeros((16,), jnp.float32)

        def zrow(k, carry):
            row = g0.at[k]
            for j in range(D // 16):
                row[pl.ds(j * 16, 16)] = zv
            return carry

        lax.fori_loop(0, K, zrow, 0)
        for g in range(640 // 16):
            zbuf[pl.ds(g * 16, 16)] = zv

        r0 = sid * RPT
        off = 0
        while off < RPT:
            sz = min(K, RPT - off)
            pltpu.sync_copy(g0.at[pl.ds(0, sz)],
                            out_acc.at[pl.ds(r0 + off, sz)])
            off += sz
        pltpu.sync_copy(zbuf.at[pl.ds(0, RPT)], s_acc.at[pl.ds(r0, RPT)])
        plsc.subcore_barrier()

        base0 = wid * EW

        def idx_copies(c, j):
            return (pltpu.make_async_copy(
                        src_hbm.at[pl.ds(base0 + c * K, K)], sidx[j],
                        sem_i[j]),
                    pltpu.make_async_copy(
                        dst_hbm.at[pl.ds(base0 + c * K, K)], didx[j],
                        sem_i[j]))

        def fire_idx(c, j):
            for cp in idx_copies(c, j):
                cp.start()

        def wait_idx(c, j):
            for cp in idx_copies(c, j):
                cp.wait()

        def att_copies(j, p):
            return (pltpu.make_async_copy(as_hbm.at[sidx[j]], asv[p],
                                          sem_a[p]),
                    pltpu.make_async_copy(ad_hbm.at[didx[j]], adv[p],
                                          sem_a[p]))

        def gather_copy(j):
            return pltpu.make_async_copy(h_hbm.at[sidx[j]], gb[j], sem_g[j])

        def fire_scatter(j):
            pltpu.async_copy(gb[j], out_acc.at[didx[j]], sem_s[j], add=True)

        def wait_scatter(j):
            pltpu.make_async_copy(gb[j], out_acc.at[didx[j]],
                                  sem_s[j]).wait()

        def step(c, j, ws=True, pi=True, ps=True):
            j1 = (j + 1) % 4
            j2 = (j + 2) % 4
            p = j % 2
            p1 = j1 % 2
            if ps:
                wait_idx(c + 1, j1)
                for cp in att_copies(j1, p1):
                    cp.start()
                gather_copy(j1).start()
            if ws:
                wait_scatter(j2)
            if pi:
                fire_idx(c + 2, j2)
            for cp in att_copies(j, p):
                cp.wait()
            for g in range(K // 16):
                sl = pl.ds(g * 16, 16)
                e = asv[p][sl] + adv[p][sl]
                wb[sl] = jnp.exp(jnp.maximum(e, 0.2 * e))
            pltpu.sync_copy(wb, s_acc.at[didx[j]], add=True)
            gather_copy(j).wait()
            gbj = gb[j]

            def scale(kk, carry):
                for u in range(2):
                    k = kk * 2 + u
                    wk = plsc.load_gather(
                        wb, [jnp.full((16,), k, jnp.int32)])
                    grow = gbj.at[k]
                    for jj in range(d_scale // 16):
                        cs = pl.ds(jj * 16, 16)
                        grow[cs] = grow[cs] * wk
                return carry

            lax.fori_loop(0, K // 2, scale, 0)
            fire_scatter(j)

        fire_idx(0, 0)
        fire_idx(1, 1)
        wait_idx(0, 0)
        for cp in att_copies(0, 0):
            cp.start()
        gather_copy(0).start()

        step(0, 0, ws=False)
        step(1, 1, ws=False)
        step(2, 2)
        step(3, 3)

        def quad(i, carry):
            c = i * 4
            step(c, 0)
            step(c + 1, 1)
            step(c + 2, 2)
            step(c + 3, 3)
            return carry

        lax.fori_loop(1, (NCHUNK - 1) // 4, quad, 0)
        step(NCHUNK - 1, 0, pi=False, ps=False)
        wait_scatter(3)
        wait_scatter(0)
        wait_idx(NCHUNK, 1)

        plsc.subcore_barrier()
        o0 = cid * NPAD + r0
        off = 0
        while off < RPT:
            sz = min(K, RPT - off)
            pltpu.sync_copy(out_acc.at[pl.ds(r0 + off, sz)],
                            g0.at[pl.ds(0, sz)])
            pltpu.sync_copy(g0.at[pl.ds(0, sz)],
                            out_hbm.at[pl.ds(o0 + off, sz)])
            off += sz
        pltpu.sync_copy(s_acc.at[pl.ds(r0, RPT)], zbuf.at[pl.ds(0, RPT)])
        pltpu.sync_copy(zbuf.at[pl.ds(0, RPT)], sout_hbm.at[pl.ds(o0, RPT)])

    return sc_agg


_sc_agg_hid = _make_sc_agg(HID)
_sc_agg_out = _make_sc_agg(OUT)


def kernel(x, edge_index, W1, att_src1, att_dst1, b1,
           W2, att_src2, att_dst2, b2):
    zk = jnp.zeros((K,), jnp.int32)
    src = jnp.concatenate([edge_index[0].astype(jnp.int32), zk])
    dst = jnp.concatenate([edge_index[1].astype(jnp.int32), zk])

    h1, av1, bv1 = _stage_a(x, W1,
                            att_src1.reshape(HID, 1), att_dst1.reshape(HID, 1))

    out1, s1p = _sc_agg_hid(src, dst, h1, av1.reshape(N), bv1.reshape(N))
    p0 = out1[:N]
    p1 = out1[NPAD:NPAD + N]
    s0 = s1p[:N].reshape(N, 1)
    s1 = s1p[NPAD:NPAD + N].reshape(N, 1)

    h2p, av2, bv2 = _stage_b(p0, p1, s0, s1, av1, bv1, h1,
                             b1.reshape(1, HID), W2,
                             att_src2.reshape(OUT, 1), att_dst2.reshape(OUT, 1))

    out2, s2p = _sc_agg_out(src, dst, h2p, av2.reshape(N), bv2.reshape(N))
    q0 = out2[:N, :OUT]
    q1 = out2[NPAD:NPAD + N, :OUT]
    t0 = s2p[:N].reshape(N, 1)
    t1 = s2p[NPAD:NPAD + N].reshape(N, 1)

    return _stage_c(q0, q1, t0, t1, av2, bv2, h2p[:, :OUT],
                    b2.reshape(1, OUT))


# speedup vs baseline: 53.0051x; 1.0028x over previous
import functools

import jax
import jax.numpy as jnp
from jax import lax
from jax.experimental import pallas as pl
from jax.experimental.pallas import tpu as pltpu
from jax.experimental.pallas import tpu_sc as plsc

N = 10000
E = 320000
D_IN = 128
HID = 128
OUT = 64

NC = 2
NS = 16
NW = NC * NS
EW = E // NW
K = 80
NCHUNK = EW // K
RPT = 632
NPAD = RPT * NS

_BR = 10000
_GRID = N // _BR


def _dot(a, b):
    return jnp.dot(a, b, preferred_element_type=jnp.float32)


def _stage_a_body(x_ref, w_ref, at_s_ref, at_d_ref, h_ref, av_ref, bv_ref):
    h = _dot(x_ref[...], w_ref[...])
    h_ref[...] = h
    av_ref[...] = _dot(h, at_s_ref[...])
    bv_ref[...] = _dot(h, at_d_ref[...])


def _stage_a(x, W1, at_s, at_d):
    return pl.pallas_call(
        _stage_a_body,
        grid=(_GRID,),
        in_specs=[
            pl.BlockSpec((_BR, D_IN), lambda i: (i, 0)),
            pl.BlockSpec((D_IN, HID), lambda i: (0, 0)),
            pl.BlockSpec((HID, 1), lambda i: (0, 0)),
            pl.BlockSpec((HID, 1), lambda i: (0, 0)),
        ],
        out_specs=[
            pl.BlockSpec((_BR, HID), lambda i: (i, 0)),
            pl.BlockSpec((_BR, 1), lambda i: (i, 0)),
            pl.BlockSpec((_BR, 1), lambda i: (i, 0)),
        ],
        out_shape=[
            jax.ShapeDtypeStruct((N, HID), jnp.float32),
            jax.ShapeDtypeStruct((N, 1), jnp.float32),
            jax.ShapeDtypeStruct((N, 1), jnp.float32),
        ],
    )(x, W1, at_s, at_d)


def _stage_b_body(p0_ref, p1_ref, s0_ref, s1_ref, av_ref, bv_ref, h1_ref,
                  b1_ref, w2_ref, a2s_ref, a2d_ref, h2_ref, av2_ref, bv2_ref):
    e = av_ref[...] + bv_ref[...]
    w = jnp.exp(jnp.maximum(e, 0.2 * e))
    num = p0_ref[...] + p1_ref[...] + w * h1_ref[...]
    den = s0_ref[...] + s1_ref[...] + w
    agg = num / den + b1_ref[...]
    x2 = jnp.where(agg > 0, agg, jnp.exp(agg) - 1.0)
    h2 = _dot(x2, w2_ref[...])
    h2_ref[...] = jnp.concatenate(
        [h2, jnp.zeros((h2.shape[0], HID - OUT), jnp.float32)], axis=1)
    av2_ref[...] = _dot(h2, a2s_ref[...])
    bv2_ref[...] = _dot(h2, a2d_ref[...])


def _stage_b(p0, p1, s0, s1, av, bv, h1, b1r, W2, a2s, a2d):
    return pl.pallas_call(
        _stage_b_body,
        grid=(_GRID,),
        in_specs=[
            pl.BlockSpec((_BR, HID), lambda i: (i, 0)),
            pl.BlockSpec((_BR, HID), lambda i: (i, 0)),
            pl.BlockSpec((_BR, 1), lambda i: (i, 0)),
            pl.BlockSpec((_BR, 1), lambda i: (i, 0)),
            pl.BlockSpec((_BR, 1), lambda i: (i, 0)),
            pl.BlockSpec((_BR, 1), lambda i: (i, 0)),
            pl.BlockSpec((_BR, HID), lambda i: (i, 0)),
            pl.BlockSpec((1, HID), lambda i: (0, 0)),
            pl.BlockSpec((HID, OUT), lambda i: (0, 0)),
            pl.BlockSpec((OUT, 1), lambda i: (0, 0)),
            pl.BlockSpec((OUT, 1), lambda i: (0, 0)),
        ],
        out_specs=[
            pl.BlockSpec((_BR, HID), lambda i: (i, 0)),
            pl.BlockSpec((_BR, 1), lambda i: (i, 0)),
            pl.BlockSpec((_BR, 1), lambda i: (i, 0)),
        ],
        out_shape=[
            jax.ShapeDtypeStruct((N, HID), jnp.float32),
            jax.ShapeDtypeStruct((N, 1), jnp.float32),
            jax.ShapeDtypeStruct((N, 1), jnp.float32),
        ],
    )(p0, p1, s0, s1, av, bv, h1, b1r, W2, a2s, a2d)


def _stage_c_body(p0_ref, p1_ref, s0_ref, s1_ref, av_ref, bv_ref, h2_ref,
                  b2_ref, y_ref):
    e = av_ref[...] + bv_ref[...]
    w = jnp.exp(jnp.maximum(e, 0.2 * e))
    num = p0_ref[...] + p1_ref[...] + w * h2_ref[...]
    den = s0_ref[...] + s1_ref[...] + w
    agg = num / den + b2_ref[...]
    m = jnp.max(agg, axis=1, keepdims=True)
    sh = agg - m
    y_ref[...] = sh - jnp.log(jnp.sum(jnp.exp(sh), axis=1, keepdims=True))


def _stage_c(p0, p1, s0, s1, av, bv, h2, b2r):
    return pl.pallas_call(
        _stage_c_body,
        grid=(_GRID,),
        in_specs=[
            pl.BlockSpec((_BR, OUT), lambda i: (i, 0)),
            pl.BlockSpec((_BR, OUT), lambda i: (i, 0)),
            pl.BlockSpec((_BR, 1), lambda i: (i, 0)),
            pl.BlockSpec((_BR, 1), lambda i: (i, 0)),
            pl.BlockSpec((_BR, 1), lambda i: (i, 0)),
            pl.BlockSpec((_BR, 1), lambda i: (i, 0)),
            pl.BlockSpec((_BR, OUT), lambda i: (i, 0)),
            pl.BlockSpec((1, OUT), lambda i: (0, 0)),
        ],
        out_specs=pl.BlockSpec((_BR, OUT), lambda i: (i, 0)),
        out_shape=jax.ShapeDtypeStruct((N, OUT), jnp.float32),
    )(p0, p1, s0, s1, av, bv, h2, b2r)


def _make_sc_agg(d_scale):
    D = HID
    mesh = plsc.VectorSubcoreMesh(core_axis_name="c", subcore_axis_name="s",
                                  num_cores=NC, num_subcores=NS)

    @functools.partial(
        pl.kernel,
        out_type=(jax.ShapeDtypeStruct((NC * NPAD, D), jnp.float32),
                  jax.ShapeDtypeStruct((NC * NPAD,), jnp.float32)),
        mesh=mesh,
        compiler_params=pltpu.CompilerParams(needs_layout_passes=False),
        scratch_types=[
            pltpu.VMEM((K, D), jnp.float32),
            pltpu.VMEM((K, D), jnp.float32),
            pltpu.VMEM((K, D), jnp.float32),
            pltpu.VMEM((K, D), jnp.float32),
            pltpu.VMEM((K,), jnp.int32),
            pltpu.VMEM((K,), jnp.int32),
            pltpu.VMEM((K,), jnp.int32),
            pltpu.VMEM((K,), jnp.int32),
            pltpu.VMEM((K,), jnp.int32),
            pltpu.VMEM((K,), jnp.int32),
            pltpu.VMEM((K,), jnp.int32),
            pltpu.VMEM((K,), jnp.int32),
            pltpu.VMEM((K,), jnp.float32),
            pltpu.VMEM((K,), jnp.float32),
            pltpu.VMEM((K,), jnp.float32),
            pltpu.VMEM((K,), jnp.float32),
            pltpu.VMEM((K,), jnp.float32),
            pltpu.VMEM((K,), jnp.float32),
            pltpu.VMEM((640,), jnp.float32),
            pltpu.VMEM_SHARED((NPAD, D), jnp.float32),
            pltpu.VMEM_SHARED((NPAD,), jnp.float32),
        ] + [pltpu.SemaphoreType.DMA] * 16,
    )
    def sc_agg(src_hbm, dst_hbm, h_hbm, as_hbm, ad_hbm,
               out_hbm, sout_hbm,
               g0, g1, g2, g3, si0, si1, si2, si3, di0, di1, di2, di3,
               av0, av1, bv0, bv1, wb0, wb1, zbuf,
               out_acc, s_acc,
               sg0, sg1, sg2, sg3, ss0, ss1, ss2, ss3,
               sia, sib, sic, sid_, sa0, sa1, sw0, sw1):
        cid = lax.axis_index("c")
        sid = lax.axis_index("s")
        wid = sid * NC + cid
        gb = (g0, g1, g2, g3)
        sidx = (si0, si1, si2, si3)
        didx = (di0, di1, di2, di3)
        asv = (av0, av1)
        adv = (bv0, bv1)
        sem_g = (sg0, sg1, sg2, sg3)
        sem_s = (ss0, ss1, ss2, ss3)
        sem_i = (sia, sib, sic, sid_)
        sem_a = (sa0, sa1)
        wbuf = (wb0, wb1)
        sem_w = (sw0, sw1)

        zv = jnp.zeros((16,), jnp.float32)

        def zrow(k, carry):
            row = g0.at[k]
            for j in range(D // 16):
                row[pl.ds(j * 16, 16)] = zv
            return carry

        lax.fori_loop(0, K, zrow, 0)
        for g in range(640 // 16):
            zbuf[pl.ds(g * 16, 16)] = zv

        r0 = sid * RPT
        off = 0
        while off < RPT:
            sz = min(K, RPT - off)
            pltpu.sync_copy(g0.at[pl.ds(0, sz)],
                            out_acc.at[pl.ds(r0 + off, sz)])
            off += sz
        pltpu.sync_copy(zbuf.at[pl.ds(0, RPT)], s_acc.at[pl.ds(r0, RPT)])
        plsc.subcore_barrier()

        base0 = wid * EW

        def idx_copies(c, j):
            return (pltpu.make_async_copy(
                        src_hbm.at[pl.ds(base0 + c * K, K)], sidx[j],
                        sem_i[j]),
                    pltpu.make_async_copy(
                        dst_hbm.at[pl.ds(base0 + c * K, K)], didx[j],
                        sem_i[j]))

        def fire_idx(c, j):
            for cp in idx_copies(c, j):
                cp.start()

        def wait_idx(c, j):
            for cp in idx_copies(c, j):
                cp.wait()

        def att_copies(j, p):
            return (pltpu.make_async_copy(as_hbm.at[sidx[j]], asv[p],
                                          sem_a[p]),
                    pltpu.make_async_copy(ad_hbm.at[didx[j]], adv[p],
                                          sem_a[p]))

        def gather_copy(j):
            return pltpu.make_async_copy(h_hbm.at[sidx[j]], gb[j], sem_g[j])

        def fire_scatter(j):
            pltpu.async_copy(gb[j], out_acc.at[didx[j]], sem_s[j], add=True)

        def wait_scatter(j):
            pltpu.make_async_copy(gb[j], out_acc.at[didx[j]],
                                  sem_s[j]).wait()

        def step(c, j, ws=True, pi=True, ps=True):
            j1 = (j + 1) % 4
            j2 = (j + 2) % 4
            p = j % 2
            p1 = j1 % 2
            if ps:
                wait_idx(c + 1, j1)
                for cp in att_copies(j1, p1):
                    cp.start()
                gather_copy(j1).start()
            if ws:
                wait_scatter(j2)
                pltpu.make_async_copy(wbuf[p], s_acc.at[didx[j2]],
                                      sem_w[p]).wait()
            if pi:
                fire_idx(c + 2, j2)
            for cp in att_copies(j, p):
                cp.wait()
            wb = wbuf[p]
            for g in range(K // 16):
                sl = pl.ds(g * 16, 16)
                e = asv[p][sl] + adv[p][sl]
                wb[sl] = jnp.exp(jnp.maximum(e, 0.2 * e))
            pltpu.async_copy(wb, s_acc.at[didx[j]], sem_w[p], add=True)
            gather_copy(j).wait()
            gbj = gb[j]

            def scale(kk, carry):
                for u in range(2):
                    k = kk * 2 + u
                    wk = plsc.load_gather(
                        wb, [jnp.full((16,), k, jnp.int32)])
                    grow = gbj.at[k]
                    for jj in range(d_scale // 16):
                        cs = pl.ds(jj * 16, 16)
                        grow[cs] = grow[cs] * wk
                return carry

            lax.fori_loop(0, K // 2, scale, 0)
            fire_scatter(j)

        fire_idx(0, 0)
        fire_idx(1, 1)
        wait_idx(0, 0)
        for cp in att_copies(0, 0):
            cp.start()
        gather_copy(0).start()

        step(0, 0, ws=False)
        step(1, 1, ws=False)
        step(2, 2)
        step(3, 3)

        def quad(i, carry):
            c = i * 4
            step(c, 0)
            step(c + 1, 1)
            step(c + 2, 2)
            step(c + 3, 3)
            return carry

        lax.fori_loop(1, (NCHUNK - 1) // 4, quad, 0)
        step(NCHUNK - 1, 0, pi=False, ps=False)
        wait_scatter(3)
        wait_scatter(0)
        pltpu.make_async_copy(wbuf[1], s_acc.at[didx[3]], sem_w[1]).wait()
        pltpu.make_async_copy(wbuf[0], s_acc.at[didx[0]], sem_w[0]).wait()
        wait_idx(NCHUNK, 1)

        plsc.subcore_barrier()
        o0 = cid * NPAD + r0
        off = 0
        while off < RPT:
            sz = min(K, RPT - off)
            pltpu.sync_copy(out_acc.at[pl.ds(r0 + off, sz)],
                            g0.at[pl.ds(0, sz)])
            pltpu.sync_copy(g0.at[pl.ds(0, sz)],
                            out_hbm.at[pl.ds(o0 + off, sz)])
            off += sz
        pltpu.sync_copy(s_acc.at[pl.ds(r0, RPT)], zbuf.at[pl.ds(0, RPT)])
        pltpu.sync_copy(zbuf.at[pl.ds(0, RPT)], sout_hbm.at[pl.ds(o0, RPT)])

    return sc_agg


_sc_agg_hid = _make_sc_agg(HID)
_sc_agg_out = _make_sc_agg(OUT)


def kernel(x, edge_index, W1, att_src1, att_dst1, b1,
           W2, att_src2, att_dst2, b2):
    zk = jnp.zeros((K,), jnp.int32)
    src = jnp.concatenate([edge_index[0].astype(jnp.int32), zk])
    dst = jnp.concatenate([edge_index[1].astype(jnp.int32), zk])

    h1, av1, bv1 = _stage_a(x, W1,
                            att_src1.reshape(HID, 1), att_dst1.reshape(HID, 1))

    out1, s1p = _sc_agg_hid(src, dst, h1, av1.reshape(N), bv1.reshape(N))
    p0 = out1[:N]
    p1 = out1[NPAD:NPAD + N]
    s0 = s1p[:N].reshape(N, 1)
    s1 = s1p[NPAD:NPAD + N].reshape(N, 1)

    h2p, av2, bv2 = _stage_b(p0, p1, s0, s1, av1, bv1, h1,
                             b1.reshape(1, HID), W2,
                             att_src2.reshape(OUT, 1), att_dst2.reshape(OUT, 1))

    out2, s2p = _sc_agg_out(src, dst, h2p, av2.reshape(N), bv2.reshape(N))
    q0 = out2[:N, :OUT]
    q1 = out2[NPAD:NPAD + N, :OUT]
    t0 = s2p[:N].reshape(N, 1)
    t1 = s2p[NPAD:NPAD + N].reshape(N, 1)

    return _stage_c(q0, q1, t0, t1, av2, bv2, h2p[:, :OUT],
                    b2.reshape(1, OUT))
